# baseline (speedup 1.0000x reference)
import functools

import jax
import jax.numpy as jnp
from jax import lax
from jax.experimental import pallas as pl
from jax.experimental.pallas import tpu as pltpu
from jax.experimental.pallas import tpu_sc as plsc

EMB = 32
NC, NS = 2, 16
NW = NC * NS
CHUNK = 128


def _sc_gather_make(batch):
    cpw = batch // (NW * CHUNK)

    @functools.partial(
        pl.kernel,
        out_type=(
            jax.ShapeDtypeStruct((NW, cpw, CHUNK, EMB), jnp.float32),
            jax.ShapeDtypeStruct((NW, cpw, CHUNK, EMB), jnp.float32),
        ),
        mesh=plsc.VectorSubcoreMesh(core_axis_name="c", subcore_axis_name="s"),
        scratch_types=[
            pltpu.VMEM((cpw, CHUNK), jnp.int32),
            pltpu.VMEM((cpw, CHUNK), jnp.int32),
            pltpu.VMEM((cpw, CHUNK, EMB), jnp.float32),
            pltpu.VMEM((cpw, CHUNK, EMB), jnp.float32),
            pltpu.SemaphoreType.DMA,
        ],
        compiler_params=pltpu.CompilerParams(use_tc_tiling_on_sc=False),
    )
    def sc_gather(uidx_hbm, iidx_hbm, utab_hbm, itab_hbm,
                  uout_hbm, iout_hbm, uidx_v, iidx_v, urows_v, irows_v, sem):
        wid = lax.axis_index("s") * NC + lax.axis_index("c")
        pltpu.sync_copy(uidx_hbm.at[pl.ds(wid * cpw, cpw)], uidx_v)
        pltpu.sync_copy(iidx_hbm.at[pl.ds(wid * cpw, cpw)], iidx_v)
        copies = []
        for j in range(cpw):
            copies.append(
                pltpu.async_copy(utab_hbm.at[uidx_v.at[j]], urows_v.at[j], sem))
            copies.append(
                pltpu.async_copy(itab_hbm.at[iidx_v.at[j]], irows_v.at[j], sem))
        for c in copies:
            c.wait()
        pltpu.sync_copy(urows_v, uout_hbm.at[wid])
        pltpu.sync_copy(irows_v, iout_hbm.at[wid])

    return sc_gather


def _mlp_body(u_ref, i_ref, w1u_ref, w1i_ref, b1_ref, w2_ref, b2_ref,
              w3_ref, b3_ref, o_ref):
    h1 = jnp.dot(u_ref[...], w1u_ref[...], preferred_element_type=jnp.float32)
    h1 = h1 + jnp.dot(i_ref[...], w1i_ref[...],
                      preferred_element_type=jnp.float32)
    h1 = jnp.maximum(h1 + b1_ref[...], 0.0)
    h2 = jnp.dot(h1, w2_ref[...], preferred_element_type=jnp.float32)
    h2 = jnp.maximum(h2 + b2_ref[...], 0.0)
    z = jnp.dot(h2, w3_ref[...], preferred_element_type=jnp.float32)
    o_ref[...] = jax.nn.sigmoid(z + b3_ref[...])


def kernel(user_input, item_input, user_table, item_table,
           W1, b1, W2, b2, W3, b3):
    batch = user_input.shape[0]
    cpw = batch // (NW * CHUNK)
    uidx = user_input.astype(jnp.int32).reshape(NW * cpw, CHUNK)
    iidx = item_input.astype(jnp.int32).reshape(NW * cpw, CHUNK)

    u_emb, i_emb = _sc_gather_make(batch)(uidx, iidx, user_table, item_table)
    u_emb = u_emb.reshape(batch, EMB)
    i_emb = i_emb.reshape(batch, EMB)

    bm = 2048
    pred = pl.pallas_call(
        _mlp_body,
        grid=(batch // bm,),
        in_specs=[
            pl.BlockSpec((bm, EMB), lambda b: (b, 0)),
            pl.BlockSpec((bm, EMB), lambda b: (b, 0)),
            pl.BlockSpec((EMB, 64), lambda b: (0, 0)),
            pl.BlockSpec((EMB, 64), lambda b: (0, 0)),
            pl.BlockSpec((1, 64), lambda b: (0, 0)),
            pl.BlockSpec((64, EMB), lambda b: (0, 0)),
            pl.BlockSpec((1, EMB), lambda b: (0, 0)),
            pl.BlockSpec((EMB, 1), lambda b: (0, 0)),
            pl.BlockSpec((1, 1), lambda b: (0, 0)),
        ],
        out_specs=pl.BlockSpec((bm, 1), lambda b: (b, 0)),
        out_shape=jax.ShapeDtypeStruct((batch, 1), jnp.float32),
    )(u_emb, i_emb, W1[:EMB], W1[EMB:], b1.reshape(1, 64),
      W2, b2.reshape(1, EMB), W3, b3.reshape(1, 1))
    return pred
